# baseline (speedup 1.0000x reference)
import jax
import jax.numpy as jnp
from jax import lax
from jax.experimental import pallas as pl
from jax.experimental.pallas import tpu as pltpu
from jax.experimental.pallas import tpu_sc as plsc

N = 10000
D = 128
E = 320000
NC = 2
NS = 16
CHUNK = 80
EDGES_PER_W = E // (NC * NS)
STEPS = EDGES_PER_W // CHUNK
ROWS_PER_S = 624
ROWS_LAST = N - (NS - 1) * ROWS_PER_S
NBUF = 3
SEGS = 5
SEG_STEPS = STEPS // SEGS

ROWS_TC = 1000
GRID_TC = N // ROWS_TC

_MESH = plsc.VectorSubcoreMesh(core_axis_name="c", subcore_axis_name="s")


def _deg_body(dst3_hbm, ones_hbm, zeros_hbm, out_hbm, didx_v, ones_v, acc_sh,
              ssem):
    c = lax.axis_index("c")
    s = lax.axis_index("s")

    @pl.when(s == 0)
    def _():
        pltpu.sync_copy(zeros_hbm, acc_sh)
    pltpu.sync_copy(dst3_hbm.at[c * NS + s], didx_v)
    pltpu.sync_copy(ones_hbm, ones_v)
    plsc.subcore_barrier()

    @pl.loop(0, STEPS // NBUF)
    def _(j):
        i0 = j * NBUF
        adds = []
        for b in range(NBUF):
            adds.append(pltpu.async_copy(
                ones_v, acc_sh.at[didx_v.at[i0 + b]], ssem.at[b], add=True))
        for b in range(NBUF):
            adds[b].wait()

    for i in range((STEPS // NBUF) * NBUF, STEPS):
        pltpu.sync_copy(ones_v, acc_sh.at[didx_v.at[i]], add=True)

    plsc.subcore_barrier()

    @pl.when(s == 0)
    def _():
        pltpu.sync_copy(acc_sh, out_hbm.at[c])


def _deg_call(dst3, ones_c, zeros_n):
    f = pl.kernel(
        _deg_body,
        out_type=jax.ShapeDtypeStruct((NC, N), jnp.float32),
        mesh=_MESH,
        scratch_types=[
            pltpu.VMEM((STEPS, CHUNK), jnp.int32),
            pltpu.VMEM((CHUNK,), jnp.float32),
            pltpu.VMEM_SHARED((N,), jnp.float32),
            pltpu.SemaphoreType.DMA((NBUF,)),
        ],
    )
    return f(dst3, ones_c, zeros_n)


def _conv_body(y_hbm, src_hbm, dst_hbm, zrows_hbm, out_hbm,
               sidx_v, didx_v, rows_v, acc_sh, gsem, ssem):
    c = lax.axis_index("c")
    s = lax.axis_index("s")

    @pl.when(s < NS - 1)
    def _():
        pltpu.sync_copy(zrows_hbm.at[pl.ds(0, ROWS_PER_S)],
                        acc_sh.at[pl.ds(s * ROWS_PER_S, ROWS_PER_S)])

    @pl.when(s == NS - 1)
    def _():
        pltpu.sync_copy(zrows_hbm,
                        acc_sh.at[pl.ds((NS - 1) * ROWS_PER_S, ROWS_LAST)])
    plsc.subcore_barrier()

    w = c * NS + s

    @pl.loop(0, SEGS)
    def _(g):
        pltpu.sync_copy(src_hbm.at[w, g], sidx_v)
        pltpu.sync_copy(dst_hbm.at[w, g], didx_v)

        @pl.loop(0, SEG_STEPS // NBUF)
        def _(j):
            i0 = j * NBUF
            gathers = []
            for b in range(NBUF):
                @pl.when(j > 0)
                def _(b=b):
                    pltpu.make_async_copy(
                        rows_v.at[b], acc_sh.at[didx_v.at[0]],
                        ssem.at[b]).wait()
                gathers.append(pltpu.async_copy(
                    y_hbm.at[sidx_v.at[i0 + b]], rows_v.at[b], gsem.at[b]))
            for b in range(NBUF):
                gathers[b].wait()
                pltpu.async_copy(
                    rows_v.at[b], acc_sh.at[didx_v.at[i0 + b]], ssem.at[b],
                    add=True)

        for b in range(NBUF):
            pltpu.make_async_copy(
                rows_v.at[b], acc_sh.at[didx_v.at[0]], ssem.at[b]).wait()

        for i in range((SEG_STEPS // NBUF) * NBUF, SEG_STEPS):
            pltpu.sync_copy(y_hbm.at[sidx_v.at[i]], rows_v.at[0])
            pltpu.sync_copy(rows_v.at[0], acc_sh.at[didx_v.at[i]], add=True)

    plsc.subcore_barrier()

    @pl.when(s < NS - 1)
    def _():
        pltpu.sync_copy(acc_sh.at[pl.ds(s * ROWS_PER_S, ROWS_PER_S)],
                        out_hbm.at[c, pl.ds(s * ROWS_PER_S, ROWS_PER_S)])

    @pl.when(s == NS - 1)
    def _():
        pltpu.sync_copy(acc_sh.at[pl.ds((NS - 1) * ROWS_PER_S, ROWS_LAST)],
                        out_hbm.at[c, pl.ds((NS - 1) * ROWS_PER_S, ROWS_LAST)])


def _conv_call(y, src3, dst3, zrows):
    f = pl.kernel(
        _conv_body,
        out_type=jax.ShapeDtypeStruct((NC, N, D), jnp.float32),
        mesh=_MESH,
        scratch_types=[
            pltpu.VMEM((SEG_STEPS, CHUNK), jnp.int32),
            pltpu.VMEM((SEG_STEPS, CHUNK), jnp.int32),
            pltpu.VMEM((NBUF, CHUNK, D), jnp.float32),
            pltpu.VMEM_SHARED((N, D), jnp.float32),
            pltpu.SemaphoreType.DMA((NBUF,)),
            pltpu.SemaphoreType.DMA((NBUF,)),
        ],
    )
    return f(y, src3, dst3, zrows)


def _scale_body(x_ref, w_ref, d0_ref, d1_ref, y_ref, dinv_ref):
    dinv = lax.rsqrt(d0_ref[...] + d1_ref[...] + 1.0)
    xw = jnp.dot(x_ref[...], w_ref[...],
                 preferred_element_type=jnp.float32,
                 precision=lax.Precision.HIGHEST)
    y_ref[...] = xw * dinv
    dinv_ref[...] = dinv


def _scale_call(x, w1, d0, d1):
    return pl.pallas_call(
        _scale_body,
        grid=(GRID_TC,),
        in_specs=[
            pl.BlockSpec((ROWS_TC, D), lambda i: (i, 0)),
            pl.BlockSpec((D, D), lambda i: (0, 0)),
            pl.BlockSpec((ROWS_TC, 1), lambda i: (i, 0)),
            pl.BlockSpec((ROWS_TC, 1), lambda i: (i, 0)),
        ],
        out_specs=[
            pl.BlockSpec((ROWS_TC, D), lambda i: (i, 0)),
            pl.BlockSpec((ROWS_TC, 1), lambda i: (i, 0)),
        ],
        out_shape=[
            jax.ShapeDtypeStruct((N, D), jnp.float32),
            jax.ShapeDtypeStruct((N, 1), jnp.float32),
        ],
    )(x, w1, d0, d1)


def _mid_body(p0_ref, p1_ref, y1_ref, dinv_ref, b1_ref, mask_ref,
              wd1_ref, bd1_ref, w2_ref, y2_ref):
    dinv = dinv_ref[...]
    h = jnp.maximum((p0_ref[...] + p1_ref[...] + y1_ref[...]) * dinv
                    + b1_ref[...], 0.0)
    hw = lax.dot_general(h, wd1_ref[...], (((1,), (1,)), ((), ())),
                         preferred_element_type=jnp.float32,
                         precision=lax.Precision.HIGHEST)
    hd = jnp.where(mask_ref[...] > 0, hw + bd1_ref[...], h)
    y2_ref[...] = jnp.dot(hd, w2_ref[...],
                          preferred_element_type=jnp.float32,
                          precision=lax.Precision.HIGHEST) * dinv


def _mid_call(p0, p1, y1, dinv, b1, maskf, wd1, bd1, w2):
    return pl.pallas_call(
        _mid_body,
        grid=(GRID_TC,),
        in_specs=[
            pl.BlockSpec((ROWS_TC, D), lambda i: (i, 0)),
            pl.BlockSpec((ROWS_TC, D), lambda i: (i, 0)),
            pl.BlockSpec((ROWS_TC, D), lambda i: (i, 0)),
            pl.BlockSpec((ROWS_TC, 1), lambda i: (i, 0)),
            pl.BlockSpec((1, D), lambda i: (0, 0)),
            pl.BlockSpec((ROWS_TC, 1), lambda i: (i, 0)),
            pl.BlockSpec((D, D), lambda i: (0, 0)),
            pl.BlockSpec((1, D), lambda i: (0, 0)),
            pl.BlockSpec((D, D), lambda i: (0, 0)),
        ],
        out_specs=pl.BlockSpec((ROWS_TC, D), lambda i: (i, 0)),
        out_shape=jax.ShapeDtypeStruct((N, D), jnp.float32),
    )(p0, p1, y1, dinv, b1, maskf, wd1, bd1, w2)


def _final_body(q0_ref, q1_ref, y2_ref, dinv_ref, b2_ref, mask_ref,
                wd2_ref, bd2_ref, out_ref):
    o = (q0_ref[...] + q1_ref[...] + y2_ref[...]) * dinv_ref[...] + b2_ref[...]
    ow = lax.dot_general(o, wd2_ref[...], (((1,), (1,)), ((), ())),
                         preferred_element_type=jnp.float32,
                         precision=lax.Precision.HIGHEST)
    out_ref[...] = jnp.where(mask_ref[...] > 0, ow + bd2_ref[...], o)


def _final_call(q0, q1, y2, dinv, b2, maskf, wd2, bd2):
    return pl.pallas_call(
        _final_body,
        grid=(GRID_TC,),
        in_specs=[
            pl.BlockSpec((ROWS_TC, D), lambda i: (i, 0)),
            pl.BlockSpec((ROWS_TC, D), lambda i: (i, 0)),
            pl.BlockSpec((ROWS_TC, D), lambda i: (i, 0)),
            pl.BlockSpec((ROWS_TC, 1), lambda i: (i, 0)),
            pl.BlockSpec((1, D), lambda i: (0, 0)),
            pl.BlockSpec((ROWS_TC, 1), lambda i: (i, 0)),
            pl.BlockSpec((D, D), lambda i: (0, 0)),
            pl.BlockSpec((1, D), lambda i: (0, 0)),
        ],
        out_specs=pl.BlockSpec((ROWS_TC, D), lambda i: (i, 0)),
        out_shape=jax.ShapeDtypeStruct((N, D), jnp.float32),
    )(q0, q1, y2, dinv, b2, maskf, wd2, bd2)


def kernel(x, edge_index, affected_mask, W1, b1, W2, b2, Wd1, bd1, Wd2, bd2):
    src = edge_index[0]
    dst = edge_index[1]
    src4 = src.reshape(NC * NS, SEGS, SEG_STEPS, CHUNK)
    dst4 = dst.reshape(NC * NS, SEGS, SEG_STEPS, CHUNK)
    dst3 = dst.reshape(NC * NS, STEPS, CHUNK)
    maskf = affected_mask.astype(jnp.float32).reshape(N, 1)
    zeros_n = jnp.zeros((N,), jnp.float32)
    zrows = jnp.zeros((ROWS_LAST, D), jnp.float32)
    ones_c = jnp.ones((CHUNK,), jnp.float32)
    b1r = b1.reshape(1, D)
    b2r = b2.reshape(1, D)
    bd1r = bd1.reshape(1, D)
    bd2r = bd2.reshape(1, D)

    degp = _deg_call(dst3, ones_c, zeros_n)
    d0 = degp[0].reshape(N, 1)
    d1 = degp[1].reshape(N, 1)

    y1, dinv = _scale_call(x, W1, d0, d1)
    p = _conv_call(y1, src4, dst4, zrows)
    y2 = _mid_call(p[0], p[1], y1, dinv, b1r, maskf, Wd1, bd1r, W2)
    q = _conv_call(y2, src4, dst4, zrows)
    return _final_call(q[0], q[1], y2, dinv, b2r, maskf, Wd2, bd2r)


# speedup vs baseline: 1.0524x; 1.0524x over previous
import jax
import jax.numpy as jnp
from jax import lax
from jax.experimental import pallas as pl
from jax.experimental.pallas import tpu as pltpu
from jax.experimental.pallas import tpu_sc as plsc

N = 10000
D = 128
E = 320000
NC = 2
NS = 16
CHUNK = 80
EDGES_PER_W = E // (NC * NS)
STEPS = EDGES_PER_W // CHUNK
ROWS_PER_S = 624
ROWS_LAST = N - (NS - 1) * ROWS_PER_S
NBUF = 4
SEGS = 5
SEG_STEPS = STEPS // SEGS

ROWS_TC = 1000
GRID_TC = N // ROWS_TC

_MESH = plsc.VectorSubcoreMesh(core_axis_name="c", subcore_axis_name="s")


def _deg_body(dst3_hbm, ones_hbm, zeros_hbm, out_hbm, didx_v, ones_v, acc_sh,
              ssem):
    c = lax.axis_index("c")
    s = lax.axis_index("s")

    @pl.when(s == 0)
    def _():
        pltpu.sync_copy(zeros_hbm, acc_sh)
    pltpu.sync_copy(dst3_hbm.at[c * NS + s], didx_v)
    pltpu.sync_copy(ones_hbm, ones_v)
    plsc.subcore_barrier()

    @pl.loop(0, STEPS // NBUF)
    def _(j):
        i0 = j * NBUF
        adds = []
        for b in range(NBUF):
            adds.append(pltpu.async_copy(
                ones_v, acc_sh.at[didx_v.at[i0 + b]], ssem.at[b], add=True))
        for b in range(NBUF):
            adds[b].wait()

    for i in range((STEPS // NBUF) * NBUF, STEPS):
        pltpu.sync_copy(ones_v, acc_sh.at[didx_v.at[i]], add=True)

    plsc.subcore_barrier()

    @pl.when(s == 0)
    def _():
        pltpu.sync_copy(acc_sh, out_hbm.at[c])


def _deg_call(dst3, ones_c, zeros_n):
    f = pl.kernel(
        _deg_body,
        out_type=jax.ShapeDtypeStruct((NC, N), jnp.float32),
        mesh=_MESH,
        scratch_types=[
            pltpu.VMEM((STEPS, CHUNK), jnp.int32),
            pltpu.VMEM((CHUNK,), jnp.float32),
            pltpu.VMEM_SHARED((N,), jnp.float32),
            pltpu.SemaphoreType.DMA((NBUF,)),
        ],
    )
    return f(dst3, ones_c, zeros_n)


def _conv_body(y_hbm, src_hbm, dst_hbm, zrows_hbm, out_hbm,
               sidx_v, didx_v, rows_v, acc_sh, gsem, ssem):
    c = lax.axis_index("c")
    s = lax.axis_index("s")

    @pl.when(s < NS - 1)
    def _():
        pltpu.sync_copy(zrows_hbm.at[pl.ds(0, ROWS_PER_S)],
                        acc_sh.at[pl.ds(s * ROWS_PER_S, ROWS_PER_S)])

    @pl.when(s == NS - 1)
    def _():
        pltpu.sync_copy(zrows_hbm,
                        acc_sh.at[pl.ds((NS - 1) * ROWS_PER_S, ROWS_LAST)])
    plsc.subcore_barrier()

    w = c * NS + s

    @pl.loop(0, SEGS)
    def _(g):
        pltpu.sync_copy(src_hbm.at[w, g], sidx_v)
        pltpu.sync_copy(dst_hbm.at[w, g], didx_v)

        @pl.loop(0, SEG_STEPS // NBUF)
        def _(j):
            i0 = j * NBUF
            gathers = []
            for b in range(NBUF):
                @pl.when(j > 0)
                def _(b=b):
                    pltpu.make_async_copy(
                        rows_v.at[b], acc_sh.at[didx_v.at[0]],
                        ssem.at[b]).wait()
                gathers.append(pltpu.async_copy(
                    y_hbm.at[sidx_v.at[i0 + b]], rows_v.at[b], gsem.at[b]))
            for b in range(NBUF):
                gathers[b].wait()
                pltpu.async_copy(
                    rows_v.at[b], acc_sh.at[didx_v.at[i0 + b]], ssem.at[b],
                    add=True)

        for b in range(NBUF):
            pltpu.make_async_copy(
                rows_v.at[b], acc_sh.at[didx_v.at[0]], ssem.at[b]).wait()

        for i in range((SEG_STEPS // NBUF) * NBUF, SEG_STEPS):
            pltpu.sync_copy(y_hbm.at[sidx_v.at[i]], rows_v.at[0])
            pltpu.sync_copy(rows_v.at[0], acc_sh.at[didx_v.at[i]], add=True)

    plsc.subcore_barrier()

    @pl.when(s < NS - 1)
    def _():
        pltpu.sync_copy(acc_sh.at[pl.ds(s * ROWS_PER_S, ROWS_PER_S)],
                        out_hbm.at[c, pl.ds(s * ROWS_PER_S, ROWS_PER_S)])

    @pl.when(s == NS - 1)
    def _():
        pltpu.sync_copy(acc_sh.at[pl.ds((NS - 1) * ROWS_PER_S, ROWS_LAST)],
                        out_hbm.at[c, pl.ds((NS - 1) * ROWS_PER_S, ROWS_LAST)])


def _conv_call(y, src3, dst3, zrows):
    f = pl.kernel(
        _conv_body,
        out_type=jax.ShapeDtypeStruct((NC, N, D), jnp.float32),
        mesh=_MESH,
        scratch_types=[
            pltpu.VMEM((SEG_STEPS, CHUNK), jnp.int32),
            pltpu.VMEM((SEG_STEPS, CHUNK), jnp.int32),
            pltpu.VMEM((NBUF, CHUNK, D), jnp.float32),
            pltpu.VMEM_SHARED((N, D), jnp.float32),
            pltpu.SemaphoreType.DMA((NBUF,)),
            pltpu.SemaphoreType.DMA((NBUF,)),
        ],
    )
    return f(y, src3, dst3, zrows)


def _scale_body(x_ref, w_ref, d0_ref, d1_ref, y_ref, dinv_ref):
    dinv = lax.rsqrt(d0_ref[...] + d1_ref[...] + 1.0)
    xw = jnp.dot(x_ref[...], w_ref[...],
                 preferred_element_type=jnp.float32,
                 precision=lax.Precision.HIGHEST)
    y_ref[...] = xw * dinv
    dinv_ref[...] = dinv


def _scale_call(x, w1, d0, d1):
    return pl.pallas_call(
        _scale_body,
        grid=(GRID_TC,),
        in_specs=[
            pl.BlockSpec((ROWS_TC, D), lambda i: (i, 0)),
            pl.BlockSpec((D, D), lambda i: (0, 0)),
            pl.BlockSpec((ROWS_TC, 1), lambda i: (i, 0)),
            pl.BlockSpec((ROWS_TC, 1), lambda i: (i, 0)),
        ],
        out_specs=[
            pl.BlockSpec((ROWS_TC, D), lambda i: (i, 0)),
            pl.BlockSpec((ROWS_TC, 1), lambda i: (i, 0)),
        ],
        out_shape=[
            jax.ShapeDtypeStruct((N, D), jnp.float32),
            jax.ShapeDtypeStruct((N, 1), jnp.float32),
        ],
    )(x, w1, d0, d1)


def _mid_body(p0_ref, p1_ref, y1_ref, dinv_ref, b1_ref, mask_ref,
              wd1_ref, bd1_ref, w2_ref, y2_ref):
    dinv = dinv_ref[...]
    h = jnp.maximum((p0_ref[...] + p1_ref[...] + y1_ref[...]) * dinv
                    + b1_ref[...], 0.0)
    hw = lax.dot_general(h, wd1_ref[...], (((1,), (1,)), ((), ())),
                         preferred_element_type=jnp.float32,
                         precision=lax.Precision.HIGHEST)
    hd = jnp.where(mask_ref[...] > 0, hw + bd1_ref[...], h)
    y2_ref[...] = jnp.dot(hd, w2_ref[...],
                          preferred_element_type=jnp.float32,
                          precision=lax.Precision.HIGHEST) * dinv


def _mid_call(p0, p1, y1, dinv, b1, maskf, wd1, bd1, w2):
    return pl.pallas_call(
        _mid_body,
        grid=(GRID_TC,),
        in_specs=[
            pl.BlockSpec((ROWS_TC, D), lambda i: (i, 0)),
            pl.BlockSpec((ROWS_TC, D), lambda i: (i, 0)),
            pl.BlockSpec((ROWS_TC, D), lambda i: (i, 0)),
            pl.BlockSpec((ROWS_TC, 1), lambda i: (i, 0)),
            pl.BlockSpec((1, D), lambda i: (0, 0)),
            pl.BlockSpec((ROWS_TC, 1), lambda i: (i, 0)),
            pl.BlockSpec((D, D), lambda i: (0, 0)),
            pl.BlockSpec((1, D), lambda i: (0, 0)),
            pl.BlockSpec((D, D), lambda i: (0, 0)),
        ],
        out_specs=pl.BlockSpec((ROWS_TC, D), lambda i: (i, 0)),
        out_shape=jax.ShapeDtypeStruct((N, D), jnp.float32),
    )(p0, p1, y1, dinv, b1, maskf, wd1, bd1, w2)


def _final_body(q0_ref, q1_ref, y2_ref, dinv_ref, b2_ref, mask_ref,
                wd2_ref, bd2_ref, out_ref):
    o = (q0_ref[...] + q1_ref[...] + y2_ref[...]) * dinv_ref[...] + b2_ref[...]
    ow = lax.dot_general(o, wd2_ref[...], (((1,), (1,)), ((), ())),
                         preferred_element_type=jnp.float32,
                         precision=lax.Precision.HIGHEST)
    out_ref[...] = jnp.where(mask_ref[...] > 0, ow + bd2_ref[...], o)


def _final_call(q0, q1, y2, dinv, b2, maskf, wd2, bd2):
    return pl.pallas_call(
        _final_body,
        grid=(GRID_TC,),
        in_specs=[
            pl.BlockSpec((ROWS_TC, D), lambda i: (i, 0)),
            pl.BlockSpec((ROWS_TC, D), lambda i: (i, 0)),
            pl.BlockSpec((ROWS_TC, D), lambda i: (i, 0)),
            pl.BlockSpec((ROWS_TC, 1), lambda i: (i, 0)),
            pl.BlockSpec((1, D), lambda i: (0, 0)),
            pl.BlockSpec((ROWS_TC, 1), lambda i: (i, 0)),
            pl.BlockSpec((D, D), lambda i: (0, 0)),
            pl.BlockSpec((1, D), lambda i: (0, 0)),
        ],
        out_specs=pl.BlockSpec((ROWS_TC, D), lambda i: (i, 0)),
        out_shape=jax.ShapeDtypeStruct((N, D), jnp.float32),
    )(q0, q1, y2, dinv, b2, maskf, wd2, bd2)


def kernel(x, edge_index, affected_mask, W1, b1, W2, b2, Wd1, bd1, Wd2, bd2):
    src = edge_index[0]
    dst = edge_index[1]
    src4 = src.reshape(NC * NS, SEGS, SEG_STEPS, CHUNK)
    dst4 = dst.reshape(NC * NS, SEGS, SEG_STEPS, CHUNK)
    dst3 = dst.reshape(NC * NS, STEPS, CHUNK)
    maskf = affected_mask.astype(jnp.float32).reshape(N, 1)
    zeros_n = jnp.zeros((N,), jnp.float32)
    zrows = jnp.zeros((ROWS_LAST, D), jnp.float32)
    ones_c = jnp.ones((CHUNK,), jnp.float32)
    b1r = b1.reshape(1, D)
    b2r = b2.reshape(1, D)
    bd1r = bd1.reshape(1, D)
    bd2r = bd2.reshape(1, D)

    degp = _deg_call(dst3, ones_c, zeros_n)
    d0 = degp[0].reshape(N, 1)
    d1 = degp[1].reshape(N, 1)

    y1, dinv = _scale_call(x, W1, d0, d1)
    p = _conv_call(y1, src4, dst4, zrows)
    y2 = _mid_call(p[0], p[1], y1, dinv, b1r, maskf, Wd1, bd1r, W2)
    q = _conv_call(y2, src4, dst4, zrows)
    return _final_call(q[0], q[1], y2, dinv, b2r, maskf, Wd2, bd2r)


# speedup vs baseline: 1.1719x; 1.1135x over previous
import jax
import jax.numpy as jnp
from jax import lax
from jax.experimental import pallas as pl
from jax.experimental.pallas import tpu as pltpu
from jax.experimental.pallas import tpu_sc as plsc

N = 10000
D = 128
E = 320000
NC = 2
NS = 16
CHUNK = 80
EDGES_PER_W = E // (NC * NS)
STEPS = EDGES_PER_W // CHUNK
ROWS_PER_S = 624
ROWS_LAST = N - (NS - 1) * ROWS_PER_S
NBUF = 4
SEGS = 5
SEG_STEPS = STEPS // SEGS

ROWS_TC = 1000
GRID_TC = N // ROWS_TC

_MESH = plsc.VectorSubcoreMesh(core_axis_name="c", subcore_axis_name="s")


def _deg_body(dst3_hbm, ones_hbm, zeros_hbm, out_hbm, didx_v, ones_v, acc_sh,
              ssem):
    c = lax.axis_index("c")
    s = lax.axis_index("s")

    @pl.when(s == 0)
    def _():
        pltpu.sync_copy(zeros_hbm, acc_sh)
    pltpu.sync_copy(dst3_hbm.at[c * NS + s], didx_v)
    pltpu.sync_copy(ones_hbm, ones_v)
    plsc.subcore_barrier()

    @pl.loop(0, STEPS // NBUF)
    def _(j):
        i0 = j * NBUF
        adds = []
        for b in range(NBUF):
            adds.append(pltpu.async_copy(
                ones_v, acc_sh.at[didx_v.at[i0 + b]], ssem.at[b], add=True))
        for b in range(NBUF):
            adds[b].wait()

    for i in range((STEPS // NBUF) * NBUF, STEPS):
        pltpu.sync_copy(ones_v, acc_sh.at[didx_v.at[i]], add=True)

    plsc.subcore_barrier()

    @pl.when(s == 0)
    def _():
        pltpu.sync_copy(acc_sh, out_hbm.at[c])


def _deg_call(dst3, ones_c, zeros_n):
    f = pl.kernel(
        _deg_body,
        out_type=jax.ShapeDtypeStruct((NC, N), jnp.float32),
        mesh=_MESH,
        scratch_types=[
            pltpu.VMEM((STEPS, CHUNK), jnp.int32),
            pltpu.VMEM((CHUNK,), jnp.float32),
            pltpu.VMEM_SHARED((N,), jnp.float32),
            pltpu.SemaphoreType.DMA((NBUF,)),
        ],
    )
    return f(dst3, ones_c, zeros_n)


def _conv_body(y_hbm, src_hbm, dst_hbm, zrows_hbm, out_hbm,
               sidx_v, didx_v, rows_v, acc_sh, gsem, ssem):
    c = lax.axis_index("c")
    s = lax.axis_index("s")

    @pl.when(s < NS - 1)
    def _():
        pltpu.sync_copy(zrows_hbm.at[pl.ds(0, ROWS_PER_S)],
                        acc_sh.at[pl.ds(s * ROWS_PER_S, ROWS_PER_S)])

    @pl.when(s == NS - 1)
    def _():
        pltpu.sync_copy(zrows_hbm,
                        acc_sh.at[pl.ds((NS - 1) * ROWS_PER_S, ROWS_LAST)])
    plsc.subcore_barrier()

    w = c * NS + s

    @pl.loop(0, SEGS)
    def _(g):
        pltpu.sync_copy(src_hbm.at[w, g], sidx_v)
        pltpu.sync_copy(dst_hbm.at[w, g], didx_v)

        @pl.loop(0, SEG_STEPS // NBUF)
        def _(j):
            i0 = j * NBUF
            gathers = []
            for b in range(NBUF):
                @pl.when(j > 0)
                def _(b=b):
                    pltpu.make_async_copy(
                        rows_v.at[b], acc_sh.at[didx_v.at[0]],
                        ssem.at[b]).wait()
                gathers.append(pltpu.async_copy(
                    y_hbm.at[sidx_v.at[i0 + b]], rows_v.at[b], gsem.at[b]))
            for b in range(NBUF):
                gathers[b].wait()
                pltpu.async_copy(
                    rows_v.at[b], acc_sh.at[didx_v.at[i0 + b]], ssem.at[b],
                    add=True)

        for b in range(NBUF):
            pltpu.make_async_copy(
                rows_v.at[b], acc_sh.at[didx_v.at[0]], ssem.at[b]).wait()

        for i in range((SEG_STEPS // NBUF) * NBUF, SEG_STEPS):
            pltpu.sync_copy(y_hbm.at[sidx_v.at[i]], rows_v.at[0])
            pltpu.sync_copy(rows_v.at[0], acc_sh.at[didx_v.at[i]], add=True)

    plsc.subcore_barrier()

    @pl.when(s < NS - 1)
    def _():
        pltpu.sync_copy(acc_sh.at[pl.ds(s * ROWS_PER_S, ROWS_PER_S)],
                        out_hbm.at[c, pl.ds(s * ROWS_PER_S, ROWS_PER_S)])

    @pl.when(s == NS - 1)
    def _():
        pltpu.sync_copy(acc_sh.at[pl.ds((NS - 1) * ROWS_PER_S, ROWS_LAST)],
                        out_hbm.at[c, pl.ds((NS - 1) * ROWS_PER_S, ROWS_LAST)])


def _conv_call(y, src3, dst3, zrows):
    f = pl.kernel(
        _conv_body,
        out_type=jax.ShapeDtypeStruct((NC, N, D), jnp.float32),
        mesh=_MESH,
        scratch_types=[
            pltpu.VMEM((SEG_STEPS, CHUNK), jnp.int32),
            pltpu.VMEM((SEG_STEPS, CHUNK), jnp.int32),
            pltpu.VMEM((NBUF, CHUNK, D), jnp.float32),
            pltpu.VMEM_SHARED((N, D), jnp.float32),
            pltpu.SemaphoreType.DMA((NBUF,)),
            pltpu.SemaphoreType.DMA((NBUF,)),
        ],
    )
    return f(y, src3, dst3, zrows)


def _scale_body(x_ref, w_ref, d0_ref, d1_ref, y_ref, dinv_ref):
    dinv = lax.rsqrt(d0_ref[...] + d1_ref[...] + 1.0)
    xw = jnp.dot(x_ref[...], w_ref[...],
                 preferred_element_type=jnp.float32)
    y_ref[...] = xw * dinv
    dinv_ref[...] = dinv


def _scale_call(x, w1, d0, d1):
    return pl.pallas_call(
        _scale_body,
        grid=(GRID_TC,),
        in_specs=[
            pl.BlockSpec((ROWS_TC, D), lambda i: (i, 0)),
            pl.BlockSpec((D, D), lambda i: (0, 0)),
            pl.BlockSpec((ROWS_TC, 1), lambda i: (i, 0)),
            pl.BlockSpec((ROWS_TC, 1), lambda i: (i, 0)),
        ],
        out_specs=[
            pl.BlockSpec((ROWS_TC, D), lambda i: (i, 0)),
            pl.BlockSpec((ROWS_TC, 1), lambda i: (i, 0)),
        ],
        out_shape=[
            jax.ShapeDtypeStruct((N, D), jnp.float32),
            jax.ShapeDtypeStruct((N, 1), jnp.float32),
        ],
    )(x, w1, d0, d1)


def _mid_body(p0_ref, p1_ref, y1_ref, dinv_ref, b1_ref, mask_ref,
              wd1_ref, bd1_ref, w2_ref, y2_ref):
    dinv = dinv_ref[...]
    h = jnp.maximum((p0_ref[0] + p1_ref[0] + y1_ref[...]) * dinv
                    + b1_ref[...], 0.0)
    hw = lax.dot_general(h, wd1_ref[...], (((1,), (1,)), ((), ())),
                         preferred_element_type=jnp.float32)
    hd = jnp.where(mask_ref[...] > 0, hw + bd1_ref[...], h)
    y2_ref[...] = jnp.dot(hd, w2_ref[...],
                          preferred_element_type=jnp.float32) * dinv


def _mid_call(p, y1, dinv, b1, maskf, wd1, bd1, w2):
    return pl.pallas_call(
        _mid_body,
        grid=(GRID_TC,),
        in_specs=[
            pl.BlockSpec((1, ROWS_TC, D), lambda i: (0, i, 0)),
            pl.BlockSpec((1, ROWS_TC, D), lambda i: (1, i, 0)),
            pl.BlockSpec((ROWS_TC, D), lambda i: (i, 0)),
            pl.BlockSpec((ROWS_TC, 1), lambda i: (i, 0)),
            pl.BlockSpec((1, D), lambda i: (0, 0)),
            pl.BlockSpec((ROWS_TC, 1), lambda i: (i, 0)),
            pl.BlockSpec((D, D), lambda i: (0, 0)),
            pl.BlockSpec((1, D), lambda i: (0, 0)),
            pl.BlockSpec((D, D), lambda i: (0, 0)),
        ],
        out_specs=pl.BlockSpec((ROWS_TC, D), lambda i: (i, 0)),
        out_shape=jax.ShapeDtypeStruct((N, D), jnp.float32),
    )(p, p, y1, dinv, b1, maskf, wd1, bd1, w2)


def _final_body(q0_ref, q1_ref, y2_ref, dinv_ref, b2_ref, mask_ref,
                wd2_ref, bd2_ref, out_ref):
    o = (q0_ref[0] + q1_ref[0] + y2_ref[...]) * dinv_ref[...] + b2_ref[...]
    ow = lax.dot_general(o, wd2_ref[...], (((1,), (1,)), ((), ())),
                         preferred_element_type=jnp.float32)
    out_ref[...] = jnp.where(mask_ref[...] > 0, ow + bd2_ref[...], o)


def _final_call(q, y2, dinv, b2, maskf, wd2, bd2):
    return pl.pallas_call(
        _final_body,
        grid=(GRID_TC,),
        in_specs=[
            pl.BlockSpec((1, ROWS_TC, D), lambda i: (0, i, 0)),
            pl.BlockSpec((1, ROWS_TC, D), lambda i: (1, i, 0)),
            pl.BlockSpec((ROWS_TC, D), lambda i: (i, 0)),
            pl.BlockSpec((ROWS_TC, 1), lambda i: (i, 0)),
            pl.BlockSpec((1, D), lambda i: (0, 0)),
            pl.BlockSpec((ROWS_TC, 1), lambda i: (i, 0)),
            pl.BlockSpec((D, D), lambda i: (0, 0)),
            pl.BlockSpec((1, D), lambda i: (0, 0)),
        ],
        out_specs=pl.BlockSpec((ROWS_TC, D), lambda i: (i, 0)),
        out_shape=jax.ShapeDtypeStruct((N, D), jnp.float32),
    )(q, q, y2, dinv, b2, maskf, wd2, bd2)


def kernel(x, edge_index, affected_mask, W1, b1, W2, b2, Wd1, bd1, Wd2, bd2):
    src = edge_index[0]
    dst = edge_index[1]
    src4 = src.reshape(NC * NS, SEGS, SEG_STEPS, CHUNK)
    dst4 = dst.reshape(NC * NS, SEGS, SEG_STEPS, CHUNK)
    dst3 = dst.reshape(NC * NS, STEPS, CHUNK)
    maskf = affected_mask.astype(jnp.float32).reshape(N, 1)
    zeros_n = jnp.zeros((N,), jnp.float32)
    zrows = jnp.zeros((ROWS_LAST, D), jnp.float32)
    ones_c = jnp.ones((CHUNK,), jnp.float32)
    b1r = b1.reshape(1, D)
    b2r = b2.reshape(1, D)
    bd1r = bd1.reshape(1, D)
    bd2r = bd2.reshape(1, D)

    degp = _deg_call(dst3, ones_c, zeros_n)
    d0 = degp[0].reshape(N, 1)
    d1 = degp[1].reshape(N, 1)

    y1, dinv = _scale_call(x, W1, d0, d1)
    p = _conv_call(y1, src4, dst4, zrows)
    y2 = _mid_call(p, y1, dinv, b1r, maskf, Wd1, bd1r, W2)
    q = _conv_call(y2, src4, dst4, zrows)
    return _final_call(q, y2, dinv, b2r, maskf, Wd2, bd2r)


# speedup vs baseline: 1.1800x; 1.0069x over previous
import jax
import jax.numpy as jnp
from jax import lax
from jax.experimental import pallas as pl
from jax.experimental.pallas import tpu as pltpu
from jax.experimental.pallas import tpu_sc as plsc

N = 10000
D = 128
E = 320000
NC = 2
NS = 16
CHUNK = 80
EDGES_PER_W = E // (NC * NS)
STEPS = EDGES_PER_W // CHUNK
ROWS_PER_S = 624
ROWS_LAST = N - (NS - 1) * ROWS_PER_S
NBUF = 4
SEGS = 5
SEG_STEPS = STEPS // SEGS

ROWS_TC = 1000
GRID_TC = N // ROWS_TC

_MESH = plsc.VectorSubcoreMesh(core_axis_name="c", subcore_axis_name="s")


def _deg_body(dst3_hbm, ones_hbm, zeros_hbm, out_hbm, didx_v, ones_v, acc_sh,
              ssem):
    c = lax.axis_index("c")
    s = lax.axis_index("s")

    @pl.when(s == 0)
    def _():
        pltpu.sync_copy(zeros_hbm, acc_sh)
    pltpu.sync_copy(dst3_hbm.at[c * NS + s], didx_v)
    pltpu.sync_copy(ones_hbm, ones_v)
    plsc.subcore_barrier()

    @pl.loop(0, STEPS // NBUF)
    def _(j):
        i0 = j * NBUF
        adds = []
        for b in range(NBUF):
            adds.append(pltpu.async_copy(
                ones_v, acc_sh.at[didx_v.at[i0 + b]], ssem.at[b], add=True))
        for b in range(NBUF):
            adds[b].wait()

    for i in range((STEPS // NBUF) * NBUF, STEPS):
        pltpu.sync_copy(ones_v, acc_sh.at[didx_v.at[i]], add=True)

    plsc.subcore_barrier()

    @pl.when(s == 0)
    def _():
        pltpu.sync_copy(acc_sh, out_hbm.at[c])


def _deg_call(dst3, ones_c, zeros_n):
    f = pl.kernel(
        _deg_body,
        out_type=jax.ShapeDtypeStruct((NC, N), jnp.float32),
        mesh=_MESH,
        scratch_types=[
            pltpu.VMEM((STEPS, CHUNK), jnp.int32),
            pltpu.VMEM((CHUNK,), jnp.float32),
            pltpu.VMEM_SHARED((N,), jnp.float32),
            pltpu.SemaphoreType.DMA((NBUF,)),
        ],
    )
    return f(dst3, ones_c, zeros_n)


def _conv_body(y_hbm, src_hbm, dst_hbm, zrows_hbm, out_hbm,
               sidx_v, didx_v, rows_v, acc_sh, gsem, ssem):
    c = lax.axis_index("c")
    s = lax.axis_index("s")

    @pl.when(s < NS - 1)
    def _():
        pltpu.sync_copy(zrows_hbm.at[pl.ds(0, ROWS_PER_S)],
                        acc_sh.at[pl.ds(s * ROWS_PER_S, ROWS_PER_S)])

    @pl.when(s == NS - 1)
    def _():
        pltpu.sync_copy(zrows_hbm,
                        acc_sh.at[pl.ds((NS - 1) * ROWS_PER_S, ROWS_LAST)])
    plsc.subcore_barrier()

    w = c * NS + s

    @pl.loop(0, SEGS)
    def _(g):
        pltpu.sync_copy(src_hbm.at[w, g], sidx_v)
        pltpu.sync_copy(dst_hbm.at[w, g], didx_v)

        @pl.loop(0, SEG_STEPS // NBUF)
        def _(j):
            i0 = j * NBUF
            gathers = []
            for b in range(NBUF):
                @pl.when(j > 0)
                def _(b=b):
                    pltpu.make_async_copy(
                        rows_v.at[b], acc_sh.at[didx_v.at[0]],
                        ssem.at[b]).wait()
                gathers.append(pltpu.async_copy(
                    y_hbm.at[sidx_v.at[i0 + b]], rows_v.at[b], gsem.at[b]))
            for b in range(NBUF):
                gathers[b].wait()
                pltpu.async_copy(
                    rows_v.at[b], acc_sh.at[didx_v.at[i0 + b]], ssem.at[b],
                    add=True)

        for b in range(NBUF):
            pltpu.make_async_copy(
                rows_v.at[b], acc_sh.at[didx_v.at[0]], ssem.at[b]).wait()

        for i in range((SEG_STEPS // NBUF) * NBUF, SEG_STEPS):
            pltpu.sync_copy(y_hbm.at[sidx_v.at[i]], rows_v.at[0])
            pltpu.sync_copy(rows_v.at[0], acc_sh.at[didx_v.at[i]], add=True)

    plsc.subcore_barrier()

    @pl.when(s < NS - 1)
    def _():
        pltpu.sync_copy(acc_sh.at[pl.ds(s * ROWS_PER_S, ROWS_PER_S)],
                        out_hbm.at[c, pl.ds(s * ROWS_PER_S, ROWS_PER_S)])

    @pl.when(s == NS - 1)
    def _():
        pltpu.sync_copy(acc_sh.at[pl.ds((NS - 1) * ROWS_PER_S, ROWS_LAST)],
                        out_hbm.at[c, pl.ds((NS - 1) * ROWS_PER_S, ROWS_LAST)])


def _conv_call(y, src3, dst3, zrows):
    f = pl.kernel(
        _conv_body,
        out_type=jax.ShapeDtypeStruct((NC, N, D), jnp.float32),
        mesh=_MESH,
        scratch_types=[
            pltpu.VMEM((SEG_STEPS, CHUNK), jnp.int32),
            pltpu.VMEM((SEG_STEPS, CHUNK), jnp.int32),
            pltpu.VMEM((NBUF, CHUNK, D), jnp.float32),
            pltpu.VMEM_SHARED((N, D), jnp.float32),
            pltpu.SemaphoreType.DMA((NBUF,)),
            pltpu.SemaphoreType.DMA((NBUF,)),
        ],
    )
    return f(y, src3, dst3, zrows)


def _scale_body(x_ref, w_ref, d0_ref, d1_ref, y_ref, dinv_ref):
    dinv = lax.rsqrt(d0_ref[0] + d1_ref[0] + 1.0)
    xw = jnp.dot(x_ref[...], w_ref[...],
                 preferred_element_type=jnp.float32)
    y_ref[...] = xw * dinv
    dinv_ref[...] = dinv


def _scale_call(x, w1, deg3):
    return pl.pallas_call(
        _scale_body,
        grid=(GRID_TC,),
        in_specs=[
            pl.BlockSpec((ROWS_TC, D), lambda i: (i, 0)),
            pl.BlockSpec((D, D), lambda i: (0, 0)),
            pl.BlockSpec((1, ROWS_TC, 1), lambda i: (0, i, 0)),
            pl.BlockSpec((1, ROWS_TC, 1), lambda i: (1, i, 0)),
        ],
        out_specs=[
            pl.BlockSpec((ROWS_TC, D), lambda i: (i, 0)),
            pl.BlockSpec((ROWS_TC, 1), lambda i: (i, 0)),
        ],
        out_shape=[
            jax.ShapeDtypeStruct((N, D), jnp.float32),
            jax.ShapeDtypeStruct((N, 1), jnp.float32),
        ],
    )(x, w1, deg3, deg3)


def _mid_body(p0_ref, p1_ref, y1_ref, dinv_ref, b1_ref, mask_ref,
              wd1_ref, bd1_ref, w2_ref, y2_ref):
    dinv = dinv_ref[...]
    h = jnp.maximum((p0_ref[0] + p1_ref[0] + y1_ref[...]) * dinv
                    + b1_ref[...], 0.0)
    hw = lax.dot_general(h, wd1_ref[...], (((1,), (1,)), ((), ())),
                         preferred_element_type=jnp.float32)
    hd = jnp.where(mask_ref[...] > 0, hw + bd1_ref[...], h)
    y2_ref[...] = jnp.dot(hd, w2_ref[...],
                          preferred_element_type=jnp.float32) * dinv


def _mid_call(p, y1, dinv, b1, maskf, wd1, bd1, w2):
    return pl.pallas_call(
        _mid_body,
        grid=(GRID_TC,),
        in_specs=[
            pl.BlockSpec((1, ROWS_TC, D), lambda i: (0, i, 0)),
            pl.BlockSpec((1, ROWS_TC, D), lambda i: (1, i, 0)),
            pl.BlockSpec((ROWS_TC, D), lambda i: (i, 0)),
            pl.BlockSpec((ROWS_TC, 1), lambda i: (i, 0)),
            pl.BlockSpec((1, D), lambda i: (0, 0)),
            pl.BlockSpec((ROWS_TC, 1), lambda i: (i, 0)),
            pl.BlockSpec((D, D), lambda i: (0, 0)),
            pl.BlockSpec((1, D), lambda i: (0, 0)),
            pl.BlockSpec((D, D), lambda i: (0, 0)),
        ],
        out_specs=pl.BlockSpec((ROWS_TC, D), lambda i: (i, 0)),
        out_shape=jax.ShapeDtypeStruct((N, D), jnp.float32),
    )(p, p, y1, dinv, b1, maskf, wd1, bd1, w2)


def _final_body(q0_ref, q1_ref, y2_ref, dinv_ref, b2_ref, mask_ref,
                wd2_ref, bd2_ref, out_ref):
    o = (q0_ref[0] + q1_ref[0] + y2_ref[...]) * dinv_ref[...] + b2_ref[...]
    ow = lax.dot_general(o, wd2_ref[...], (((1,), (1,)), ((), ())),
                         preferred_element_type=jnp.float32)
    out_ref[...] = jnp.where(mask_ref[...] > 0, ow + bd2_ref[...], o)


def _final_call(q, y2, dinv, b2, maskf, wd2, bd2):
    return pl.pallas_call(
        _final_body,
        grid=(GRID_TC,),
        in_specs=[
            pl.BlockSpec((1, ROWS_TC, D), lambda i: (0, i, 0)),
            pl.BlockSpec((1, ROWS_TC, D), lambda i: (1, i, 0)),
            pl.BlockSpec((ROWS_TC, D), lambda i: (i, 0)),
            pl.BlockSpec((ROWS_TC, 1), lambda i: (i, 0)),
            pl.BlockSpec((1, D), lambda i: (0, 0)),
            pl.BlockSpec((ROWS_TC, 1), lambda i: (i, 0)),
            pl.BlockSpec((D, D), lambda i: (0, 0)),
            pl.BlockSpec((1, D), lambda i: (0, 0)),
        ],
        out_specs=pl.BlockSpec((ROWS_TC, D), lambda i: (i, 0)),
        out_shape=jax.ShapeDtypeStruct((N, D), jnp.float32),
    )(q, q, y2, dinv, b2, maskf, wd2, bd2)


def kernel(x, edge_index, affected_mask, W1, b1, W2, b2, Wd1, bd1, Wd2, bd2):
    src = edge_index[0]
    dst = edge_index[1]
    src4 = src.reshape(NC * NS, SEGS, SEG_STEPS, CHUNK)
    dst4 = dst.reshape(NC * NS, SEGS, SEG_STEPS, CHUNK)
    dst3 = dst.reshape(NC * NS, STEPS, CHUNK)
    maskf = affected_mask.astype(jnp.float32).reshape(N, 1)
    zeros_n = jnp.zeros((N,), jnp.float32)
    zrows = jnp.zeros((ROWS_LAST, D), jnp.float32)
    ones_c = jnp.ones((CHUNK,), jnp.float32)
    b1r = b1.reshape(1, D)
    b2r = b2.reshape(1, D)
    bd1r = bd1.reshape(1, D)
    bd2r = bd2.reshape(1, D)

    degp = _deg_call(dst3, ones_c, zeros_n)
    y1, dinv = _scale_call(x, W1, degp.reshape(NC, N, 1))
    p = _conv_call(y1, src4, dst4, zrows)
    y2 = _mid_call(p, y1, dinv, b1r, maskf, Wd1, bd1r, W2)
    q = _conv_call(y2, src4, dst4, zrows)
    return _final_call(q, y2, dinv, b2r, maskf, Wd2, bd2r)


# speedup vs baseline: 1.2084x; 1.0241x over previous
import jax
import jax.numpy as jnp
from jax import lax
from jax.experimental import pallas as pl
from jax.experimental.pallas import tpu as pltpu
from jax.experimental.pallas import tpu_sc as plsc

N = 10000
D = 128
E = 320000
NC = 2
NS = 16
CHUNK = 80
EDGES_PER_W = E // (NC * NS)
STEPS = EDGES_PER_W // CHUNK
ROWS_PER_S = 624
ROWS_LAST = N - (NS - 1) * ROWS_PER_S
NBUF = 4
SEGS = 5
SEG_STEPS = STEPS // SEGS

ROWS_TC = 2000
GRID_TC = N // ROWS_TC

_MESH = plsc.VectorSubcoreMesh(core_axis_name="c", subcore_axis_name="s")


def _deg_body(dst3_hbm, ones_hbm, zeros_hbm, out_hbm, didx_v, ones_v, acc_sh,
              ssem):
    c = lax.axis_index("c")
    s = lax.axis_index("s")

    @pl.when(s == 0)
    def _():
        pltpu.sync_copy(zeros_hbm, acc_sh)
    pltpu.sync_copy(dst3_hbm.at[c * NS + s], didx_v)
    pltpu.sync_copy(ones_hbm, ones_v)
    plsc.subcore_barrier()

    @pl.loop(0, STEPS // NBUF)
    def _(j):
        i0 = j * NBUF
        adds = []
        for b in range(NBUF):
            adds.append(pltpu.async_copy(
                ones_v, acc_sh.at[didx_v.at[i0 + b]], ssem.at[b], add=True))
        for b in range(NBUF):
            adds[b].wait()

    for i in range((STEPS // NBUF) * NBUF, STEPS):
        pltpu.sync_copy(ones_v, acc_sh.at[didx_v.at[i]], add=True)

    plsc.subcore_barrier()

    @pl.when(s == 0)
    def _():
        pltpu.sync_copy(acc_sh, out_hbm.at[c])


def _deg_call(dst3, ones_c, zeros_n):
    f = pl.kernel(
        _deg_body,
        out_type=jax.ShapeDtypeStruct((NC, N), jnp.float32),
        mesh=_MESH,
        scratch_types=[
            pltpu.VMEM((STEPS, CHUNK), jnp.int32),
            pltpu.VMEM((CHUNK,), jnp.float32),
            pltpu.VMEM_SHARED((N,), jnp.float32),
            pltpu.SemaphoreType.DMA((NBUF,)),
        ],
    )
    return f(dst3, ones_c, zeros_n)


def _conv_body(y_hbm, src_hbm, dst_hbm, zrows_hbm, out_hbm,
               sidx_v, didx_v, rows_v, acc_sh, gsem, ssem, zsem):
    c = lax.axis_index("c")
    s = lax.axis_index("s")

    @pl.when(s < NS - 1)
    def _():
        pltpu.async_copy(zrows_hbm.at[pl.ds(0, ROWS_PER_S)],
                         acc_sh.at[pl.ds(s * ROWS_PER_S, ROWS_PER_S)], zsem)

    @pl.when(s == NS - 1)
    def _():
        pltpu.async_copy(zrows_hbm,
                         acc_sh.at[pl.ds((NS - 1) * ROWS_PER_S, ROWS_LAST)],
                         zsem)

    w = c * NS + s

    @pl.loop(0, SEGS)
    def _(g):
        pltpu.sync_copy(src_hbm.at[w, g], sidx_v)
        pltpu.sync_copy(dst_hbm.at[w, g], didx_v)

        @pl.when(g == 0)
        def _():
            @pl.when(s < NS - 1)
            def _():
                pltpu.make_async_copy(
                    zrows_hbm.at[pl.ds(0, ROWS_PER_S)],
                    acc_sh.at[pl.ds(s * ROWS_PER_S, ROWS_PER_S)], zsem).wait()

            @pl.when(s == NS - 1)
            def _():
                pltpu.make_async_copy(
                    zrows_hbm,
                    acc_sh.at[pl.ds((NS - 1) * ROWS_PER_S, ROWS_LAST)],
                    zsem).wait()
            plsc.subcore_barrier()

        @pl.loop(0, SEG_STEPS // NBUF)
        def _(j):
            i0 = j * NBUF
            gathers = []
            for b in range(NBUF):
                @pl.when(j > 0)
                def _(b=b):
                    pltpu.make_async_copy(
                        rows_v.at[b], acc_sh.at[didx_v.at[0]],
                        ssem.at[b]).wait()
                gathers.append(pltpu.async_copy(
                    y_hbm.at[sidx_v.at[i0 + b]], rows_v.at[b], gsem.at[b]))
            for b in range(NBUF):
                gathers[b].wait()
                pltpu.async_copy(
                    rows_v.at[b], acc_sh.at[didx_v.at[i0 + b]], ssem.at[b],
                    add=True)

        for b in range(NBUF):
            pltpu.make_async_copy(
                rows_v.at[b], acc_sh.at[didx_v.at[0]], ssem.at[b]).wait()

        for i in range((SEG_STEPS // NBUF) * NBUF, SEG_STEPS):
            pltpu.sync_copy(y_hbm.at[sidx_v.at[i]], rows_v.at[0])
            pltpu.sync_copy(rows_v.at[0], acc_sh.at[didx_v.at[i]], add=True)

    plsc.subcore_barrier()

    @pl.when(s < NS - 1)
    def _():
        pltpu.sync_copy(acc_sh.at[pl.ds(s * ROWS_PER_S, ROWS_PER_S)],
                        out_hbm.at[c, pl.ds(s * ROWS_PER_S, ROWS_PER_S)])

    @pl.when(s == NS - 1)
    def _():
        pltpu.sync_copy(acc_sh.at[pl.ds((NS - 1) * ROWS_PER_S, ROWS_LAST)],
                        out_hbm.at[c, pl.ds((NS - 1) * ROWS_PER_S, ROWS_LAST)])


def _conv_call(y, src3, dst3, zrows):
    f = pl.kernel(
        _conv_body,
        out_type=jax.ShapeDtypeStruct((NC, N, D), jnp.float32),
        mesh=_MESH,
        scratch_types=[
            pltpu.VMEM((SEG_STEPS, CHUNK), jnp.int32),
            pltpu.VMEM((SEG_STEPS, CHUNK), jnp.int32),
            pltpu.VMEM((NBUF, CHUNK, D), jnp.float32),
            pltpu.VMEM_SHARED((N, D), jnp.float32),
            pltpu.SemaphoreType.DMA((NBUF,)),
            pltpu.SemaphoreType.DMA((NBUF,)),
            pltpu.SemaphoreType.DMA,
        ],
    )
    return f(y, src3, dst3, zrows)


def _scale_body(x_ref, w_ref, d0_ref, d1_ref, y_ref, dinv_ref):
    dinv = lax.rsqrt(d0_ref[0] + d1_ref[0] + 1.0)
    xw = jnp.dot(x_ref[...], w_ref[...],
                 preferred_element_type=jnp.float32)
    y_ref[...] = xw * dinv
    dinv_ref[...] = dinv


def _scale_call(x, w1, deg3):
    return pl.pallas_call(
        _scale_body,
        grid=(GRID_TC,),
        in_specs=[
            pl.BlockSpec((ROWS_TC, D), lambda i: (i, 0)),
            pl.BlockSpec((D, D), lambda i: (0, 0)),
            pl.BlockSpec((1, ROWS_TC, 1), lambda i: (0, i, 0)),
            pl.BlockSpec((1, ROWS_TC, 1), lambda i: (1, i, 0)),
        ],
        out_specs=[
            pl.BlockSpec((ROWS_TC, D), lambda i: (i, 0)),
            pl.BlockSpec((ROWS_TC, 1), lambda i: (i, 0)),
        ],
        out_shape=[
            jax.ShapeDtypeStruct((N, D), jnp.float32),
            jax.ShapeDtypeStruct((N, 1), jnp.float32),
        ],
    )(x, w1, deg3, deg3)


def _mid_body(p0_ref, p1_ref, y1_ref, dinv_ref, b1_ref, mask_ref,
              wd1_ref, bd1_ref, w2_ref, y2_ref):
    dinv = dinv_ref[...]
    h = jnp.maximum((p0_ref[0] + p1_ref[0] + y1_ref[...]) * dinv
                    + b1_ref[...], 0.0)
    hw = lax.dot_general(h, wd1_ref[...], (((1,), (1,)), ((), ())),
                         preferred_element_type=jnp.float32)
    hd = jnp.where(mask_ref[...] > 0, hw + bd1_ref[...], h)
    y2_ref[...] = jnp.dot(hd, w2_ref[...],
                          preferred_element_type=jnp.float32) * dinv


def _mid_call(p, y1, dinv, b1, maskf, wd1, bd1, w2):
    return pl.pallas_call(
        _mid_body,
        grid=(GRID_TC,),
        in_specs=[
            pl.BlockSpec((1, ROWS_TC, D), lambda i: (0, i, 0)),
            pl.BlockSpec((1, ROWS_TC, D), lambda i: (1, i, 0)),
            pl.BlockSpec((ROWS_TC, D), lambda i: (i, 0)),
            pl.BlockSpec((ROWS_TC, 1), lambda i: (i, 0)),
            pl.BlockSpec((1, D), lambda i: (0, 0)),
            pl.BlockSpec((ROWS_TC, 1), lambda i: (i, 0)),
            pl.BlockSpec((D, D), lambda i: (0, 0)),
            pl.BlockSpec((1, D), lambda i: (0, 0)),
            pl.BlockSpec((D, D), lambda i: (0, 0)),
        ],
        out_specs=pl.BlockSpec((ROWS_TC, D), lambda i: (i, 0)),
        out_shape=jax.ShapeDtypeStruct((N, D), jnp.float32),
    )(p, p, y1, dinv, b1, maskf, wd1, bd1, w2)


def _final_body(q0_ref, q1_ref, y2_ref, dinv_ref, b2_ref, mask_ref,
                wd2_ref, bd2_ref, out_ref):
    o = (q0_ref[0] + q1_ref[0] + y2_ref[...]) * dinv_ref[...] + b2_ref[...]
    ow = lax.dot_general(o, wd2_ref[...], (((1,), (1,)), ((), ())),
                         preferred_element_type=jnp.float32)
    out_ref[...] = jnp.where(mask_ref[...] > 0, ow + bd2_ref[...], o)


def _final_call(q, y2, dinv, b2, maskf, wd2, bd2):
    return pl.pallas_call(
        _final_body,
        grid=(GRID_TC,),
        in_specs=[
            pl.BlockSpec((1, ROWS_TC, D), lambda i: (0, i, 0)),
            pl.BlockSpec((1, ROWS_TC, D), lambda i: (1, i, 0)),
            pl.BlockSpec((ROWS_TC, D), lambda i: (i, 0)),
            pl.BlockSpec((ROWS_TC, 1), lambda i: (i, 0)),
            pl.BlockSpec((1, D), lambda i: (0, 0)),
            pl.BlockSpec((ROWS_TC, 1), lambda i: (i, 0)),
            pl.BlockSpec((D, D), lambda i: (0, 0)),
            pl.BlockSpec((1, D), lambda i: (0, 0)),
        ],
        out_specs=pl.BlockSpec((ROWS_TC, D), lambda i: (i, 0)),
        out_shape=jax.ShapeDtypeStruct((N, D), jnp.float32),
    )(q, q, y2, dinv, b2, maskf, wd2, bd2)


def kernel(x, edge_index, affected_mask, W1, b1, W2, b2, Wd1, bd1, Wd2, bd2):
    src = edge_index[0]
    dst = edge_index[1]
    src4 = src.reshape(NC * NS, SEGS, SEG_STEPS, CHUNK)
    dst4 = dst.reshape(NC * NS, SEGS, SEG_STEPS, CHUNK)
    dst3 = dst.reshape(NC * NS, STEPS, CHUNK)
    maskf = affected_mask.astype(jnp.float32).reshape(N, 1)
    zeros_n = jnp.zeros((N,), jnp.float32)
    zrows = jnp.zeros((ROWS_LAST, D), jnp.float32)
    ones_c = jnp.ones((CHUNK,), jnp.float32)
    b1r = b1.reshape(1, D)
    b2r = b2.reshape(1, D)
    bd1r = bd1.reshape(1, D)
    bd2r = bd2.reshape(1, D)

    degp = _deg_call(dst3, ones_c, zeros_n)
    y1, dinv = _scale_call(x, W1, degp.reshape(NC, N, 1))
    p = _conv_call(y1, src4, dst4, zrows)
    y2 = _mid_call(p, y1, dinv, b1r, maskf, Wd1, bd1r, W2)
    q = _conv_call(y2, src4, dst4, zrows)
    return _final_call(q, y2, dinv, b2r, maskf, Wd2, bd2r)


# speedup vs baseline: 1.2110x; 1.0022x over previous
import jax
import jax.numpy as jnp
from jax import lax
from jax.experimental import pallas as pl
from jax.experimental.pallas import tpu as pltpu
from jax.experimental.pallas import tpu_sc as plsc

N = 10000
D = 128
E = 320000
NC = 2
NS = 16
CHUNK = 80
EDGES_PER_W = E // (NC * NS)
STEPS = EDGES_PER_W // CHUNK
ROWS_PER_S = 624
ROWS_LAST = N - (NS - 1) * ROWS_PER_S
NBUF = 4
SEGS = 5
SEG_STEPS = STEPS // SEGS

ROWS_TC = 5000
GRID_TC = N // ROWS_TC

_MESH = plsc.VectorSubcoreMesh(core_axis_name="c", subcore_axis_name="s")


def _deg_body(dst3_hbm, ones_hbm, zeros_hbm, out_hbm, didx_v, ones_v, acc_sh,
              ssem):
    c = lax.axis_index("c")
    s = lax.axis_index("s")

    @pl.when(s == 0)
    def _():
        pltpu.sync_copy(zeros_hbm, acc_sh)
    pltpu.sync_copy(dst3_hbm.at[c * NS + s], didx_v)
    pltpu.sync_copy(ones_hbm, ones_v)
    plsc.subcore_barrier()

    @pl.loop(0, STEPS // NBUF)
    def _(j):
        i0 = j * NBUF
        adds = []
        for b in range(NBUF):
            adds.append(pltpu.async_copy(
                ones_v, acc_sh.at[didx_v.at[i0 + b]], ssem.at[b], add=True))
        for b in range(NBUF):
            adds[b].wait()

    for i in range((STEPS // NBUF) * NBUF, STEPS):
        pltpu.sync_copy(ones_v, acc_sh.at[didx_v.at[i]], add=True)

    plsc.subcore_barrier()

    @pl.when(s == 0)
    def _():
        pltpu.sync_copy(acc_sh, out_hbm.at[c])


def _deg_call(dst3, ones_c, zeros_n):
    f = pl.kernel(
        _deg_body,
        out_type=jax.ShapeDtypeStruct((NC, N), jnp.float32),
        mesh=_MESH,
        scratch_types=[
            pltpu.VMEM((STEPS, CHUNK), jnp.int32),
            pltpu.VMEM((CHUNK,), jnp.float32),
            pltpu.VMEM_SHARED((N,), jnp.float32),
            pltpu.SemaphoreType.DMA((NBUF,)),
        ],
    )
    return f(dst3, ones_c, zeros_n)


def _conv_body(y_hbm, src_hbm, dst_hbm, zrows_hbm, out_hbm,
               sidx_v, didx_v, rows_v, acc_sh, gsem, ssem, zsem):
    c = lax.axis_index("c")
    s = lax.axis_index("s")

    @pl.when(s < NS - 1)
    def _():
        pltpu.async_copy(zrows_hbm.at[pl.ds(0, ROWS_PER_S)],
                         acc_sh.at[pl.ds(s * ROWS_PER_S, ROWS_PER_S)], zsem)

    @pl.when(s == NS - 1)
    def _():
        pltpu.async_copy(zrows_hbm,
                         acc_sh.at[pl.ds((NS - 1) * ROWS_PER_S, ROWS_LAST)],
                         zsem)

    w = c * NS + s

    @pl.loop(0, SEGS)
    def _(g):
        pltpu.sync_copy(src_hbm.at[w, g], sidx_v)
        pltpu.sync_copy(dst_hbm.at[w, g], didx_v)

        @pl.when(g == 0)
        def _():
            @pl.when(s < NS - 1)
            def _():
                pltpu.make_async_copy(
                    zrows_hbm.at[pl.ds(0, ROWS_PER_S)],
                    acc_sh.at[pl.ds(s * ROWS_PER_S, ROWS_PER_S)], zsem).wait()

            @pl.when(s == NS - 1)
            def _():
                pltpu.make_async_copy(
                    zrows_hbm,
                    acc_sh.at[pl.ds((NS - 1) * ROWS_PER_S, ROWS_LAST)],
                    zsem).wait()
            plsc.subcore_barrier()

        @pl.loop(0, SEG_STEPS // NBUF)
        def _(j):
            i0 = j * NBUF
            gathers = []
            for b in range(NBUF):
                @pl.when(j > 0)
                def _(b=b):
                    pltpu.make_async_copy(
                        rows_v.at[b], acc_sh.at[didx_v.at[0]],
                        ssem.at[b]).wait()
                gathers.append(pltpu.async_copy(
                    y_hbm.at[sidx_v.at[i0 + b]], rows_v.at[b], gsem.at[b]))
            for b in range(NBUF):
                gathers[b].wait()
                pltpu.async_copy(
                    rows_v.at[b], acc_sh.at[didx_v.at[i0 + b]], ssem.at[b],
                    add=True)

        for b in range(NBUF):
            pltpu.make_async_copy(
                rows_v.at[b], acc_sh.at[didx_v.at[0]], ssem.at[b]).wait()

        for i in range((SEG_STEPS // NBUF) * NBUF, SEG_STEPS):
            pltpu.sync_copy(y_hbm.at[sidx_v.at[i]], rows_v.at[0])
            pltpu.sync_copy(rows_v.at[0], acc_sh.at[didx_v.at[i]], add=True)

    plsc.subcore_barrier()

    @pl.when(s < NS - 1)
    def _():
        pltpu.sync_copy(acc_sh.at[pl.ds(s * ROWS_PER_S, ROWS_PER_S)],
                        out_hbm.at[c, pl.ds(s * ROWS_PER_S, ROWS_PER_S)])

    @pl.when(s == NS - 1)
    def _():
        pltpu.sync_copy(acc_sh.at[pl.ds((NS - 1) * ROWS_PER_S, ROWS_LAST)],
                        out_hbm.at[c, pl.ds((NS - 1) * ROWS_PER_S, ROWS_LAST)])


def _conv_call(y, src3, dst3, zrows):
    f = pl.kernel(
        _conv_body,
        out_type=jax.ShapeDtypeStruct((NC, N, D), jnp.float32),
        mesh=_MESH,
        scratch_types=[
            pltpu.VMEM((SEG_STEPS, CHUNK), jnp.int32),
            pltpu.VMEM((SEG_STEPS, CHUNK), jnp.int32),
            pltpu.VMEM((NBUF, CHUNK, D), jnp.float32),
            pltpu.VMEM_SHARED((N, D), jnp.float32),
            pltpu.SemaphoreType.DMA((NBUF,)),
            pltpu.SemaphoreType.DMA((NBUF,)),
            pltpu.SemaphoreType.DMA,
        ],
    )
    return f(y, src3, dst3, zrows)


def _scale_body(x_ref, w_ref, d0_ref, d1_ref, y_ref, dinv_ref):
    dinv = lax.rsqrt(d0_ref[0] + d1_ref[0] + 1.0)
    xw = jnp.dot(x_ref[...], w_ref[...],
                 preferred_element_type=jnp.float32)
    y_ref[...] = xw * dinv
    dinv_ref[...] = dinv


def _scale_call(x, w1, deg3):
    return pl.pallas_call(
        _scale_body,
        grid=(GRID_TC,),
        in_specs=[
            pl.BlockSpec((ROWS_TC, D), lambda i: (i, 0)),
            pl.BlockSpec((D, D), lambda i: (0, 0)),
            pl.BlockSpec((1, ROWS_TC, 1), lambda i: (0, i, 0)),
            pl.BlockSpec((1, ROWS_TC, 1), lambda i: (1, i, 0)),
        ],
        out_specs=[
            pl.BlockSpec((ROWS_TC, D), lambda i: (i, 0)),
            pl.BlockSpec((ROWS_TC, 1), lambda i: (i, 0)),
        ],
        out_shape=[
            jax.ShapeDtypeStruct((N, D), jnp.float32),
            jax.ShapeDtypeStruct((N, 1), jnp.float32),
        ],
    )(x, w1, deg3, deg3)


def _mid_body(p0_ref, p1_ref, y1_ref, dinv_ref, b1_ref, mask_ref,
              wd1_ref, bd1_ref, w2_ref, y2_ref):
    dinv = dinv_ref[...]
    h = jnp.maximum((p0_ref[0] + p1_ref[0] + y1_ref[...]) * dinv
                    + b1_ref[...], 0.0)
    hw = lax.dot_general(h, wd1_ref[...], (((1,), (1,)), ((), ())),
                         preferred_element_type=jnp.float32)
    hd = jnp.where(mask_ref[...] > 0, hw + bd1_ref[...], h)
    y2_ref[...] = jnp.dot(hd, w2_ref[...],
                          preferred_element_type=jnp.float32) * dinv


def _mid_call(p, y1, dinv, b1, maskf, wd1, bd1, w2):
    return pl.pallas_call(
        _mid_body,
        grid=(GRID_TC,),
        in_specs=[
            pl.BlockSpec((1, ROWS_TC, D), lambda i: (0, i, 0)),
            pl.BlockSpec((1, ROWS_TC, D), lambda i: (1, i, 0)),
            pl.BlockSpec((ROWS_TC, D), lambda i: (i, 0)),
            pl.BlockSpec((ROWS_TC, 1), lambda i: (i, 0)),
            pl.BlockSpec((1, D), lambda i: (0, 0)),
            pl.BlockSpec((ROWS_TC, 1), lambda i: (i, 0)),
            pl.BlockSpec((D, D), lambda i: (0, 0)),
            pl.BlockSpec((1, D), lambda i: (0, 0)),
            pl.BlockSpec((D, D), lambda i: (0, 0)),
        ],
        out_specs=pl.BlockSpec((ROWS_TC, D), lambda i: (i, 0)),
        out_shape=jax.ShapeDtypeStruct((N, D), jnp.float32),
    )(p, p, y1, dinv, b1, maskf, wd1, bd1, w2)


def _final_body(q0_ref, q1_ref, y2_ref, dinv_ref, b2_ref, mask_ref,
                wd2_ref, bd2_ref, out_ref):
    o = (q0_ref[0] + q1_ref[0] + y2_ref[...]) * dinv_ref[...] + b2_ref[...]
    ow = lax.dot_general(o, wd2_ref[...], (((1,), (1,)), ((), ())),
                         preferred_element_type=jnp.float32)
    out_ref[...] = jnp.where(mask_ref[...] > 0, ow + bd2_ref[...], o)


def _final_call(q, y2, dinv, b2, maskf, wd2, bd2):
    return pl.pallas_call(
        _final_body,
        grid=(GRID_TC,),
        in_specs=[
            pl.BlockSpec((1, ROWS_TC, D), lambda i: (0, i, 0)),
            pl.BlockSpec((1, ROWS_TC, D), lambda i: (1, i, 0)),
            pl.BlockSpec((ROWS_TC, D), lambda i: (i, 0)),
            pl.BlockSpec((ROWS_TC, 1), lambda i: (i, 0)),
            pl.BlockSpec((1, D), lambda i: (0, 0)),
            pl.BlockSpec((ROWS_TC, 1), lambda i: (i, 0)),
            pl.BlockSpec((D, D), lambda i: (0, 0)),
            pl.BlockSpec((1, D), lambda i: (0, 0)),
        ],
        out_specs=pl.BlockSpec((ROWS_TC, D), lambda i: (i, 0)),
        out_shape=jax.ShapeDtypeStruct((N, D), jnp.float32),
    )(q, q, y2, dinv, b2, maskf, wd2, bd2)


def kernel(x, edge_index, affected_mask, W1, b1, W2, b2, Wd1, bd1, Wd2, bd2):
    src = edge_index[0]
    dst = edge_index[1]
    src4 = src.reshape(NC * NS, SEGS, SEG_STEPS, CHUNK)
    dst4 = dst.reshape(NC * NS, SEGS, SEG_STEPS, CHUNK)
    dst3 = dst.reshape(NC * NS, STEPS, CHUNK)
    maskf = affected_mask.astype(jnp.float32).reshape(N, 1)
    zeros_n = jnp.zeros((N,), jnp.float32)
    zrows = jnp.zeros((ROWS_LAST, D), jnp.float32)
    ones_c = jnp.ones((CHUNK,), jnp.float32)
    b1r = b1.reshape(1, D)
    b2r = b2.reshape(1, D)
    bd1r = bd1.reshape(1, D)
    bd2r = bd2.reshape(1, D)

    degp = _deg_call(dst3, ones_c, zeros_n)
    y1, dinv = _scale_call(x, W1, degp.reshape(NC, N, 1))
    p = _conv_call(y1, src4, dst4, zrows)
    y2 = _mid_call(p, y1, dinv, b1r, maskf, Wd1, bd1r, W2)
    q = _conv_call(y2, src4, dst4, zrows)
    return _final_call(q, y2, dinv, b2r, maskf, Wd2, bd2r)


# speedup vs baseline: 1.2364x; 1.0209x over previous
import jax
import jax.numpy as jnp
from jax import lax
from jax.experimental import pallas as pl
from jax.experimental.pallas import tpu as pltpu
from jax.experimental.pallas import tpu_sc as plsc

N = 10000
D = 128
E = 320000
NC = 2
NS = 16
CHUNK = 80
EDGES_PER_W = E // (NC * NS)
STEPS = EDGES_PER_W // CHUNK
ROWS_PER_S = 624
ROWS_LAST = N - (NS - 1) * ROWS_PER_S
NBUF = 4
SEGS = 5
SEG_STEPS = STEPS // SEGS

ROWS_TC = 5000
GRID_TC = N // ROWS_TC

_MESH = plsc.VectorSubcoreMesh(core_axis_name="c", subcore_axis_name="s")


def _deg_body(dst3_hbm, ones_hbm, zeros_hbm, out_hbm, didx_v, ones_v, acc_sh,
              ssem):
    c = lax.axis_index("c")
    s = lax.axis_index("s")

    @pl.when(s == 0)
    def _():
        pltpu.sync_copy(zeros_hbm, acc_sh)
    pltpu.sync_copy(dst3_hbm.at[c * NS + s], didx_v)
    pltpu.sync_copy(ones_hbm, ones_v)
    plsc.subcore_barrier()

    @pl.loop(0, STEPS // NBUF)
    def _(j):
        i0 = j * NBUF
        adds = []
        for b in range(NBUF):
            adds.append(pltpu.async_copy(
                ones_v, acc_sh.at[didx_v.at[i0 + b]], ssem.at[b], add=True))
        for b in range(NBUF):
            adds[b].wait()

    for i in range((STEPS // NBUF) * NBUF, STEPS):
        pltpu.sync_copy(ones_v, acc_sh.at[didx_v.at[i]], add=True)

    plsc.subcore_barrier()

    @pl.when(s == 0)
    def _():
        pltpu.sync_copy(acc_sh, out_hbm.at[c])


def _deg_call(dst3, ones_c, zeros_n):
    f = pl.kernel(
        _deg_body,
        out_type=jax.ShapeDtypeStruct((NC, N), jnp.float32),
        mesh=_MESH,
        scratch_types=[
            pltpu.VMEM((STEPS, CHUNK), jnp.int32),
            pltpu.VMEM((CHUNK,), jnp.float32),
            pltpu.VMEM_SHARED((N,), jnp.float32),
            pltpu.SemaphoreType.DMA((NBUF,)),
        ],
    )
    return f(dst3, ones_c, zeros_n)


def _conv_body(y_hbm, src_hbm, dst_hbm, zrows_hbm, out_hbm,
               sidx_v, didx_v, rows_v, acc_sh, gsem, ssem, zsem):
    c = lax.axis_index("c")
    s = lax.axis_index("s")

    @pl.when(s < NS - 1)
    def _():
        pltpu.async_copy(zrows_hbm.at[pl.ds(0, ROWS_PER_S)],
                         acc_sh.at[pl.ds(s * ROWS_PER_S, ROWS_PER_S)], zsem)

    @pl.when(s == NS - 1)
    def _():
        pltpu.async_copy(zrows_hbm,
                         acc_sh.at[pl.ds((NS - 1) * ROWS_PER_S, ROWS_LAST)],
                         zsem)

    w = c * NS + s

    @pl.loop(0, SEGS)
    def _(g):
        pltpu.sync_copy(src_hbm.at[w, g], sidx_v)
        pltpu.sync_copy(dst_hbm.at[w, g], didx_v)

        @pl.when(g == 0)
        def _():
            @pl.when(s < NS - 1)
            def _():
                pltpu.make_async_copy(
                    zrows_hbm.at[pl.ds(0, ROWS_PER_S)],
                    acc_sh.at[pl.ds(s * ROWS_PER_S, ROWS_PER_S)], zsem).wait()

            @pl.when(s == NS - 1)
            def _():
                pltpu.make_async_copy(
                    zrows_hbm,
                    acc_sh.at[pl.ds((NS - 1) * ROWS_PER_S, ROWS_LAST)],
                    zsem).wait()
            plsc.subcore_barrier()

        @pl.loop(0, SEG_STEPS // NBUF)
        def _(j):
            i0 = j * NBUF
            gathers = []
            for b in range(NBUF):
                @pl.when(j > 0)
                def _(b=b):
                    pltpu.make_async_copy(
                        rows_v.at[b], acc_sh.at[didx_v.at[0]],
                        ssem.at[b]).wait()
                gathers.append(pltpu.async_copy(
                    y_hbm.at[sidx_v.at[i0 + b]], rows_v.at[b], gsem.at[b]))
            for b in range(NBUF):
                gathers[b].wait()
                pltpu.async_copy(
                    rows_v.at[b], acc_sh.at[didx_v.at[i0 + b]], ssem.at[b],
                    add=True)

        ti = (SEG_STEPS // NBUF) * NBUF
        pltpu.make_async_copy(
            rows_v.at[0], acc_sh.at[didx_v.at[0]], ssem.at[0]).wait()
        tg = pltpu.async_copy(y_hbm.at[sidx_v.at[ti]], rows_v.at[0], gsem.at[0])
        for b in range(1, NBUF):
            pltpu.make_async_copy(
                rows_v.at[b], acc_sh.at[didx_v.at[0]], ssem.at[b]).wait()
        tg.wait()
        pltpu.sync_copy(rows_v.at[0], acc_sh.at[didx_v.at[ti]], add=True)

    plsc.subcore_barrier()

    @pl.when(s < NS - 1)
    def _():
        pltpu.sync_copy(acc_sh.at[pl.ds(s * ROWS_PER_S, ROWS_PER_S)],
                        out_hbm.at[c, pl.ds(s * ROWS_PER_S, ROWS_PER_S)])

    @pl.when(s == NS - 1)
    def _():
        pltpu.sync_copy(acc_sh.at[pl.ds((NS - 1) * ROWS_PER_S, ROWS_LAST)],
                        out_hbm.at[c, pl.ds((NS - 1) * ROWS_PER_S, ROWS_LAST)])


def _conv_call(y, src3, dst3, zrows):
    f = pl.kernel(
        _conv_body,
        out_type=jax.ShapeDtypeStruct((NC, N, D), jnp.float32),
        mesh=_MESH,
        scratch_types=[
            pltpu.VMEM((SEG_STEPS, CHUNK), jnp.int32),
            pltpu.VMEM((SEG_STEPS, CHUNK), jnp.int32),
            pltpu.VMEM((NBUF, CHUNK, D), jnp.float32),
            pltpu.VMEM_SHARED((N, D), jnp.float32),
            pltpu.SemaphoreType.DMA((NBUF,)),
            pltpu.SemaphoreType.DMA((NBUF,)),
            pltpu.SemaphoreType.DMA,
        ],
    )
    return f(y, src3, dst3, zrows)


def _scale_body(x_ref, w_ref, d0_ref, d1_ref, y_ref, dinv_ref):
    dinv = lax.rsqrt(d0_ref[0] + d1_ref[0] + 1.0)
    xw = jnp.dot(x_ref[...], w_ref[...],
                 preferred_element_type=jnp.float32)
    y_ref[...] = xw * dinv
    dinv_ref[...] = dinv


def _scale_call(x, w1, deg3):
    return pl.pallas_call(
        _scale_body,
        grid=(GRID_TC,),
        in_specs=[
            pl.BlockSpec((ROWS_TC, D), lambda i: (i, 0)),
            pl.BlockSpec((D, D), lambda i: (0, 0)),
            pl.BlockSpec((1, ROWS_TC, 1), lambda i: (0, i, 0)),
            pl.BlockSpec((1, ROWS_TC, 1), lambda i: (1, i, 0)),
        ],
        out_specs=[
            pl.BlockSpec((ROWS_TC, D), lambda i: (i, 0)),
            pl.BlockSpec((ROWS_TC, 1), lambda i: (i, 0)),
        ],
        out_shape=[
            jax.ShapeDtypeStruct((N, D), jnp.float32),
            jax.ShapeDtypeStruct((N, 1), jnp.float32),
        ],
    )(x, w1, deg3, deg3)


def _mid_body(p0_ref, p1_ref, y1_ref, dinv_ref, b1_ref, mask_ref,
              wd1_ref, bd1_ref, w2_ref, y2_ref):
    dinv = dinv_ref[...]
    h = jnp.maximum((p0_ref[0] + p1_ref[0] + y1_ref[...]) * dinv
                    + b1_ref[...], 0.0)
    hw = lax.dot_general(h, wd1_ref[...], (((1,), (1,)), ((), ())),
                         preferred_element_type=jnp.float32)
    hd = jnp.where(mask_ref[...] > 0, hw + bd1_ref[...], h)
    y2_ref[...] = jnp.dot(hd, w2_ref[...],
                          preferred_element_type=jnp.float32) * dinv


def _mid_call(p, y1, dinv, b1, maskf, wd1, bd1, w2):
    return pl.pallas_call(
        _mid_body,
        grid=(GRID_TC,),
        in_specs=[
            pl.BlockSpec((1, ROWS_TC, D), lambda i: (0, i, 0)),
            pl.BlockSpec((1, ROWS_TC, D), lambda i: (1, i, 0)),
            pl.BlockSpec((ROWS_TC, D), lambda i: (i, 0)),
            pl.BlockSpec((ROWS_TC, 1), lambda i: (i, 0)),
            pl.BlockSpec((1, D), lambda i: (0, 0)),
            pl.BlockSpec((ROWS_TC, 1), lambda i: (i, 0)),
            pl.BlockSpec((D, D), lambda i: (0, 0)),
            pl.BlockSpec((1, D), lambda i: (0, 0)),
            pl.BlockSpec((D, D), lambda i: (0, 0)),
        ],
        out_specs=pl.BlockSpec((ROWS_TC, D), lambda i: (i, 0)),
        out_shape=jax.ShapeDtypeStruct((N, D), jnp.float32),
    )(p, p, y1, dinv, b1, maskf, wd1, bd1, w2)


def _final_body(q0_ref, q1_ref, y2_ref, dinv_ref, b2_ref, mask_ref,
                wd2_ref, bd2_ref, out_ref):
    o = (q0_ref[0] + q1_ref[0] + y2_ref[...]) * dinv_ref[...] + b2_ref[...]
    ow = lax.dot_general(o, wd2_ref[...], (((1,), (1,)), ((), ())),
                         preferred_element_type=jnp.float32)
    out_ref[...] = jnp.where(mask_ref[...] > 0, ow + bd2_ref[...], o)


def _final_call(q, y2, dinv, b2, maskf, wd2, bd2):
    return pl.pallas_call(
        _final_body,
        grid=(GRID_TC,),
        in_specs=[
            pl.BlockSpec((1, ROWS_TC, D), lambda i: (0, i, 0)),
            pl.BlockSpec((1, ROWS_TC, D), lambda i: (1, i, 0)),
            pl.BlockSpec((ROWS_TC, D), lambda i: (i, 0)),
            pl.BlockSpec((ROWS_TC, 1), lambda i: (i, 0)),
            pl.BlockSpec((1, D), lambda i: (0, 0)),
            pl.BlockSpec((ROWS_TC, 1), lambda i: (i, 0)),
            pl.BlockSpec((D, D), lambda i: (0, 0)),
            pl.BlockSpec((1, D), lambda i: (0, 0)),
        ],
        out_specs=pl.BlockSpec((ROWS_TC, D), lambda i: (i, 0)),
        out_shape=jax.ShapeDtypeStruct((N, D), jnp.float32),
    )(q, q, y2, dinv, b2, maskf, wd2, bd2)


def kernel(x, edge_index, affected_mask, W1, b1, W2, b2, Wd1, bd1, Wd2, bd2):
    src = edge_index[0]
    dst = edge_index[1]
    src4 = src.reshape(NC * NS, SEGS, SEG_STEPS, CHUNK)
    dst4 = dst.reshape(NC * NS, SEGS, SEG_STEPS, CHUNK)
    dst3 = dst.reshape(NC * NS, STEPS, CHUNK)
    maskf = affected_mask.astype(jnp.float32).reshape(N, 1)
    zeros_n = jnp.zeros((N,), jnp.float32)
    zrows = jnp.zeros((ROWS_LAST, D), jnp.float32)
    ones_c = jnp.ones((CHUNK,), jnp.float32)
    b1r = b1.reshape(1, D)
    b2r = b2.reshape(1, D)
    bd1r = bd1.reshape(1, D)
    bd2r = bd2.reshape(1, D)

    degp = _deg_call(dst3, ones_c, zeros_n)
    y1, dinv = _scale_call(x, W1, degp.reshape(NC, N, 1))
    p = _conv_call(y1, src4, dst4, zrows)
    y2 = _mid_call(p, y1, dinv, b1r, maskf, Wd1, bd1r, W2)
    q = _conv_call(y2, src4, dst4, zrows)
    return _final_call(q, y2, dinv, b2r, maskf, Wd2, bd2r)
